# Initial kernel scaffold; baseline (speedup 1.0000x reference)
import functools

import jax
import jax.numpy as jnp
from jax import lax
from jax.experimental import pallas as pl
from jax.experimental.pallas import tpu as pltpu
from jax.experimental.pallas import tpu_sc as plsc

_B = 16384
_T = 10
_F = 5
_NC = 2
_NS = 16
_NW = _NC * _NS
_BPW = _B // _NW
_L = 16
_CHUNKS = _BPW // _L


def _sc_body(idx_hbm, tab_hbm, out_hbm, idx_v, tab_v, out_v):
    wid = lax.axis_index("s") * _NC + lax.axis_index("c")
    base = wid * _BPW
    pltpu.sync_copy(tab_hbm, tab_v)
    pltpu.sync_copy(idx_hbm.at[:, pl.ds(base, _BPW)], idx_v)

    def chunk(c, carry):
        col = c * _L
        acc0 = [None] * _F
        acc1 = [None] * _F
        for t in range(_T):
            idx5 = idx_v[t, pl.ds(col, _L)] * _F
            for f in range(_F):
                v = plsc.load_gather(tab_v, [idx5 + f])
                if t < 5:
                    acc0[f] = v if acc0[f] is None else acc0[f] * v
                else:
                    acc1[f] = v if acc1[f] is None else acc1[f] * v
        for f in range(_F):
            out_v[f, pl.ds(col, _L)] = (acc0[f] + acc1[f]) * 0.5
        return carry

    lax.fori_loop(0, _CHUNKS, chunk, 0)
    pltpu.sync_copy(out_v, out_hbm.at[:, pl.ds(base, _BPW)])


_sc_kernel = functools.partial(
    pl.kernel,
    out_type=jax.ShapeDtypeStruct((_F, _B), jnp.float32),
    mesh=plsc.VectorSubcoreMesh(core_axis_name="c", subcore_axis_name="s"),
    scratch_types=[
        pltpu.VMEM((_T, _BPW), jnp.int32),
        pltpu.VMEM((128,), jnp.float32),
        pltpu.VMEM((_F, _BPW), jnp.float32),
    ],
)(_sc_body)


def kernel(inputs, table):
    idx_t = inputs.T.astype(jnp.int32)
    tab_flat = jnp.pad(table.reshape(-1), (0, 28))
    out_t = _sc_kernel(idx_t, tab_flat)
    return out_t.T


# speedup vs baseline: 31.6141x; 31.6141x over previous
import functools

import jax
import jax.numpy as jnp
from jax import lax
from jax.experimental import pallas as pl
from jax.experimental.pallas import tpu as pltpu
from jax.experimental.pallas import tpu_sc as plsc

_B = 16384
_T = 10
_F = 5
_NC = 2
_NS = 16
_NW = _NC * _NS
_BPW = _B // _NW
_L = 16
_CHUNKS = _BPW // _L


def _sc_body(idx_hbm, tab_hbm, out_hbm, idx_v, tab_v, out_v):
    wid = lax.axis_index("s") * _NC + lax.axis_index("c")
    base = wid * _BPW
    pltpu.sync_copy(tab_hbm, tab_v)
    pltpu.sync_copy(idx_hbm.at[:, pl.ds(base, _BPW)], idx_v)

    def chunk(c, carry):
        col = c * _L
        acc0 = [None] * _F
        acc1 = [None] * _F
        for t in range(_T):
            idx5 = idx_v[t, pl.ds(col, _L)] * _F
            for f in range(_F):
                v = plsc.load_gather(tab_v, [idx5 + f])
                if t < 5:
                    acc0[f] = v if acc0[f] is None else acc0[f] * v
                else:
                    acc1[f] = v if acc1[f] is None else acc1[f] * v
        for f in range(_F):
            out_v[f, pl.ds(col, _L)] = (acc0[f] + acc1[f]) * 0.5
        return carry

    lax.fori_loop(0, _CHUNKS, chunk, 0)
    pltpu.sync_copy(out_v, out_hbm.at[:, pl.ds(base, _BPW)])


_sc_kernel = functools.partial(
    pl.kernel,
    out_type=jax.ShapeDtypeStruct((_F, _B), jnp.float32),
    mesh=plsc.VectorSubcoreMesh(core_axis_name="c", subcore_axis_name="s"),
    compiler_params=pltpu.CompilerParams(needs_layout_passes=False),
    scratch_types=[
        pltpu.VMEM((_T, _BPW), jnp.int32),
        pltpu.VMEM((128,), jnp.float32),
        pltpu.VMEM((_F, _BPW), jnp.float32),
    ],
)(_sc_body)


def kernel(inputs, table):
    idx_t = inputs.T.astype(jnp.int32)
    tab_flat = jnp.pad(table.reshape(-1), (0, 28))
    out_t = _sc_kernel(idx_t, tab_flat)
    return out_t.T
